# Initial kernel scaffold; baseline (speedup 1.0000x reference)
import jax
import jax.numpy as jnp
from jax.experimental import pallas as pl


def kernel(x, pos, batch, W1a, b1a, W1b, b1b, W2a, b2a, W2b, b2b, W3a, b3a, W3b, b3b, Wr, br):
    raise NotImplementedError("write your pallas kernel here")


# speedup vs baseline: 69.5256x; 69.5256x over previous
import functools

import jax
import jax.numpy as jnp
import numpy as np
from jax import lax
from jax.experimental import pallas as pl
from jax.experimental.pallas import tpu as pltpu
from jax.experimental.pallas import tpu_sc as plsc

CH = 64
NBATCH = 8
PADB = 127
_NW = 32


def _round_up(x, m):
    return (x + m - 1) // m * m


def _sc_gather(table, idx):
    V, D = table.shape
    B = idx.shape[0]
    b_per_w = B // _NW
    mesh = plsc.VectorSubcoreMesh(core_axis_name="c", subcore_axis_name="s")

    @functools.partial(
        pl.kernel,
        out_type=jax.ShapeDtypeStruct((B, D), jnp.float32),
        mesh=mesh,
        scratch_types=[
            pltpu.VMEM((b_per_w,), jnp.int32),
            pltpu.VMEM((b_per_w, D), jnp.float32),
            pltpu.SemaphoreType.DMA,
        ],
        compiler_params=pltpu.CompilerParams(use_tc_tiling_on_sc=False),
    )
    def gk(table_hbm, idx_hbm, out_hbm, idx_v, rows_v, sem):
        wid = lax.axis_index("s") * 2 + lax.axis_index("c")
        base = wid * b_per_w
        pltpu.sync_copy(idx_hbm.at[pl.ds(base, b_per_w)], idx_v)
        pltpu.async_copy(table_hbm.at[idx_v], rows_v, sem).wait()
        pltpu.sync_copy(rows_v, out_hbm.at[pl.ds(base, b_per_w)])

    return gk(table, idx)


def _knn_kernel(k, R, Np, pr_ref, br_ref, pT_ref, bc_ref, out_ref):
    colio = lax.broadcasted_iota(jnp.int32, (R, Np), 1)
    d = None
    for c in range(3):
        diff = pr_ref[:, c : c + 1] - pT_ref[c : c + 1, :]
        sq = diff * diff
        d = sq if d is None else d + sq
    inf = jnp.float32(jnp.inf)
    same = br_ref[:, 0:1] == bc_ref[...]
    dcur = jnp.where(same, d, inf)
    taken = colio < 0
    big = jnp.int32(2**30)
    for j in range(k):
        mn = jnp.min(dcur, axis=1, keepdims=True)
        cand = (dcur == mn) & jnp.logical_not(taken)
        amin = jnp.min(jnp.where(cand, colio, big), axis=1, keepdims=True)
        out_ref[:, j : j + 1] = amin
        hit = colio == amin
        taken = taken | hit
        dcur = jnp.where(hit, inf, dcur)


def _knn_call(pos_pad, posT, batch_rows, batch_cols, k, R=256):
    Np = pos_pad.shape[0]
    nt = Np // R
    return pl.pallas_call(
        functools.partial(_knn_kernel, k, R, Np),
        grid=(nt,),
        in_specs=[
            pl.BlockSpec((R, 4), lambda i: (i, 0)),
            pl.BlockSpec((R, 1), lambda i: (i, 0)),
            pl.BlockSpec((4, Np), lambda i: (0, 0)),
            pl.BlockSpec((1, Np), lambda i: (0, 0)),
        ],
        out_specs=pl.BlockSpec((R, 8), lambda i: (i, 0)),
        out_shape=jax.ShapeDtypeStruct((Np, 8), jnp.int32),
    )(pos_pad, batch_rows, posT, batch_cols)


def _fps_kernel(C, Ms, mm_ref, n_ref, px_ref, py_ref, pz_ref, sel_ref):
    px = px_ref[...]
    py = py_ref[...]
    pz = pz_ref[...]
    colio = lax.broadcasted_iota(jnp.int32, (NBATCH, C), 1)
    valid = colio < n_ref[...]
    ninf = jnp.float32(-jnp.inf)
    big = jnp.int32(2**30)
    eye = lax.broadcasted_iota(jnp.int32, (NBATCH, NBATCH), 0) == lax.broadcasted_iota(
        jnp.int32, (NBATCH, NBATCH), 1
    )

    def dist_to(sx, sy, sz):
        dx = px - sx
        dy = py - sy
        dz = pz - sz
        return (dx * dx + dy * dy) + dz * dz

    d0 = dist_to(px[:, 0:1], py[:, 0:1], pz[:, 0:1])
    dd = jnp.where(valid, d0, ninf)
    sel_ref[0:1, :] = jnp.zeros((1, NBATCH), jnp.int32)

    def body(r, dd):
        mx = jnp.max(dd, axis=1, keepdims=True)
        nxt = jnp.min(jnp.where(dd == mx, colio, big), axis=1, keepdims=True)
        row = jnp.sum(
            jnp.where(eye, jnp.broadcast_to(nxt, (NBATCH, NBATCH)), 0),
            axis=0,
            keepdims=True,
        )
        sel_ref[pl.ds(r, 1), :] = row
        hit = colio == nxt
        sx = jnp.max(jnp.where(hit, px, ninf), axis=1, keepdims=True)
        sy = jnp.max(jnp.where(hit, py, ninf), axis=1, keepdims=True)
        sz = jnp.max(jnp.where(hit, pz, ninf), axis=1, keepdims=True)
        return jnp.minimum(dd, dist_to(sx, sy, sz))

    lax.fori_loop(1, mm_ref[0], body, dd)


def _fps_call(px, py, pz, n_seg, max_m, C):
    Ms = C // 2 + 128
    return pl.pallas_call(
        functools.partial(_fps_kernel, C, Ms),
        in_specs=[
            pl.BlockSpec(memory_space=pltpu.SMEM),
            pl.BlockSpec((NBATCH, 1), lambda: (0, 0)),
            pl.BlockSpec((NBATCH, C), lambda: (0, 0)),
            pl.BlockSpec((NBATCH, C), lambda: (0, 0)),
            pl.BlockSpec((NBATCH, C), lambda: (0, 0)),
        ],
        out_specs=pl.BlockSpec((Ms, NBATCH), lambda: (0, 0)),
        out_shape=jax.ShapeDtypeStruct((Ms, NBATCH), jnp.int32),
    )(max_m, n_seg, px, py, pz)


def _layer_kernel(k, tdst_ref, tsrc_ref, wd_ref, ws_ref, ba_ref, wb_ref, bb_ref, out_ref):
    a = (
        jnp.dot(tdst_ref[...], wd_ref[...], preferred_element_type=jnp.float32)
        + ba_ref[...]
    )
    wb = wb_ref[...]
    bb = bb_ref[...]
    acc = None
    for j in range(k):
        bj = jnp.dot(tsrc_ref[j], ws_ref[...], preferred_element_type=jnp.float32)
        m = jnp.dot(jnp.maximum(a + bj, 0.0), wb, preferred_element_type=jnp.float32) + bb
        acc = m if acc is None else jnp.maximum(acc, m)
    out_ref[...] = jnp.maximum(acc, 0.0)


def _layer_call(table, src_gath, wd, ws, ba, wb, bb, k, R=256):
    Np, D = table.shape
    nt = Np // R
    return pl.pallas_call(
        functools.partial(_layer_kernel, k),
        grid=(nt,),
        in_specs=[
            pl.BlockSpec((R, D), lambda i: (i, 0)),
            pl.BlockSpec((k, R, D), lambda i: (0, i, 0)),
            pl.BlockSpec((D, CH), lambda i: (0, 0)),
            pl.BlockSpec((D, CH), lambda i: (0, 0)),
            pl.BlockSpec((1, CH), lambda i: (0, 0)),
            pl.BlockSpec((CH, CH), lambda i: (0, 0)),
            pl.BlockSpec((1, CH), lambda i: (0, 0)),
        ],
        out_specs=pl.BlockSpec((R, CH), lambda i: (i, 0)),
        out_shape=jax.ShapeDtypeStruct((Np, CH), jnp.float32),
    )(table, src_gath, wd, ws, ba, wb, bb)


def _head_kernel(h_ref, b3_ref, wr_ref, br_ref, out_ref):
    ninf = jnp.float32(-jnp.inf)
    h = h_ref[...]
    rows = []
    for b in range(NBATCH):
        mb = b3_ref[...] == b
        rows.append(jnp.max(jnp.where(mb, h, ninf), axis=0, keepdims=True))
    g = jnp.concatenate(rows, axis=0)
    out_ref[...] = jnp.dot(g, wr_ref[...], preferred_element_type=jnp.float32) + br_ref[...]


def _head_call(h3, b3, wr_pad, br_pad):
    Np = h3.shape[0]
    return pl.pallas_call(
        _head_kernel,
        in_specs=[
            pl.BlockSpec((Np, CH), lambda: (0, 0)),
            pl.BlockSpec((Np, 1), lambda: (0, 0)),
            pl.BlockSpec((CH, 8), lambda: (0, 0)),
            pl.BlockSpec((1, 8), lambda: (0, 0)),
        ],
        out_specs=pl.BlockSpec((NBATCH, 8), lambda: (0, 0)),
        out_shape=jax.ShapeDtypeStruct((NBATCH, 8), jnp.float32),
    )(h3, b3, wr_pad, br_pad)


def _pack_weights(wa, hdim, D):
    wd = jnp.zeros((D, CH), jnp.float32)
    ws = jnp.zeros((D, CH), jnp.float32)
    wpos = wa[2 * hdim : 2 * hdim + 3]
    if hdim == 3:
        wd = wd.at[0:3].set(wa[0:3] - wpos)
        ws = ws.at[0:3].set(wa[3:6] + wpos)
    else:
        wd = wd.at[0:CH].set(wa[0:CH]).at[CH : CH + 3].set(-wpos)
        ws = ws.at[0:CH].set(wa[CH : 2 * CH]).at[CH : CH + 3].set(wpos)
    return wd, ws


def _fps_level(posT_seg, seg_start, counts, C, M_pad):
    m = (counts + 1) // 2
    max_m = jnp.max(m).astype(jnp.int32).reshape((1,))
    wins = jnp.stack(
        [
            lax.dynamic_slice(posT_seg, (0, seg_start[b]), (3, C))
            for b in range(NBATCH)
        ]
    )
    sel = _fps_call(
        wins[:, 0, :], wins[:, 1, :], wins[:, 2, :], counts.reshape(NBATCH, 1), max_m, C
    )
    Ms = sel.shape[0]
    gsel = sel + seg_start[None, :].astype(jnp.int32)
    ar = jnp.arange(Ms, dtype=jnp.int32)
    offs = jnp.cumsum(m) - m
    dest = jnp.where(ar[:, None] < m[None, :], offs[None, :] + ar[:, None], M_pad)
    idx = (
        jnp.zeros((M_pad,), jnp.int32)
        .at[dest.reshape(-1)]
        .set(gsel.reshape(-1), mode="drop")
    )
    return idx, m


def kernel(x, pos, batch, W1a, b1a, W1b, b1b, W2a, b2a, W2b, b2b, W3a, b3a, W3b, b3b, Wr, br):
    N = pos.shape[0]
    Np1 = _round_up(N, 256)
    batch = batch.astype(jnp.int32)

    pos_pad = jnp.zeros((Np1, 4), jnp.float32).at[:N, :3].set(pos)
    posT = pos_pad[:, :3].T
    posT4 = jnp.zeros((4, Np1), jnp.float32).at[:3].set(posT)
    bpad = jnp.full((Np1,), PADB, jnp.int32).at[:N].set(batch)
    counts1 = jnp.sum(
        batch[None, :] == jnp.arange(NBATCH, dtype=jnp.int32)[:, None], axis=1
    ).astype(jnp.int32)
    seg1 = jnp.cumsum(counts1) - counts1

    nbr1 = _knn_call(pos_pad, posT4, bpad.reshape(Np1, 1), bpad.reshape(1, Np1), k=6)
    src1 = nbr1[:, :6].T.reshape(-1)
    table1 = jnp.zeros((Np1, 16), jnp.float32).at[:, :3].set(pos_pad[:, :3])
    g1 = _sc_gather(table1, src1).reshape(6, Np1, 16)
    wd1, ws1 = _pack_weights(W1a, 3, 16)
    h1 = _layer_call(table1, g1, wd1, ws1, b1a.reshape(1, CH), W1b, bb=b1b.reshape(1, CH), k=6)

    posT2x = jnp.zeros((4, 2 * Np1), jnp.float32).at[:, :Np1].set(posT4)
    M_pad1 = (N + 1) // 2 + NBATCH
    i1, m1 = _fps_level(posT2x, seg1, counts1, Np1, M_pad1)
    Np2 = _round_up(M_pad1, 256)
    i1p = jnp.zeros((Np2,), jnp.int32).at[:M_pad1].set(i1)

    ntab1 = jnp.concatenate([h1, table1], axis=1)
    table2 = _sc_gather(ntab1, i1p)
    pos2 = table2[:, CH : CH + 3]
    cum1 = jnp.cumsum(m1)
    b2 = jnp.searchsorted(cum1, jnp.arange(M_pad1, dtype=jnp.int32), side="right").astype(jnp.int32)
    b2pad = jnp.full((Np2,), PADB, jnp.int32).at[:M_pad1].set(b2)

    pos2_pad = jnp.zeros((Np2, 4), jnp.float32).at[:, :3].set(pos2)
    pos2T4 = jnp.zeros((4, Np2), jnp.float32).at[:3].set(pos2.T)
    nbr2 = _knn_call(pos2_pad, pos2T4, b2pad.reshape(Np2, 1), b2pad.reshape(1, Np2), k=4)
    src2 = nbr2[:, :4].T.reshape(-1)
    g2 = _sc_gather(table2, src2).reshape(4, Np2, 80)
    wd2, ws2 = _pack_weights(W2a, CH, 80)
    h2 = _layer_call(table2, g2, wd2, ws2, b2a.reshape(1, CH), W2b, bb=b2b.reshape(1, CH), k=4)

    pos2T2x = jnp.zeros((4, 2 * Np2), jnp.float32).at[:, :Np2].set(pos2T4)
    M_pad2 = (M_pad1 + 1) // 2 + NBATCH
    i2, m2 = _fps_level(pos2T2x, jnp.cumsum(m1) - m1, m1, Np2, M_pad2)
    Np3 = _round_up(M_pad2, 256)
    i2p = jnp.zeros((Np3,), jnp.int32).at[:M_pad2].set(i2)

    ntab2 = jnp.concatenate([h2, table2[:, CH : CH + 16]], axis=1)
    table3 = _sc_gather(ntab2, i2p)
    pos3 = table3[:, CH : CH + 3]
    cum2 = jnp.cumsum(m2)
    b3 = jnp.searchsorted(cum2, jnp.arange(M_pad2, dtype=jnp.int32), side="right").astype(jnp.int32)
    b3pad = jnp.full((Np3,), PADB, jnp.int32).at[:M_pad2].set(b3)

    pos3_pad = jnp.zeros((Np3, 4), jnp.float32).at[:, :3].set(pos3)
    pos3T4 = jnp.zeros((4, Np3), jnp.float32).at[:3].set(pos3.T)
    nbr3 = _knn_call(pos3_pad, pos3T4, b3pad.reshape(Np3, 1), b3pad.reshape(1, Np3), k=3)
    src3 = nbr3[:, :3].T.reshape(-1)
    g3 = _sc_gather(table3, src3).reshape(3, Np3, 80)
    wd3, ws3 = _pack_weights(W3a, CH, 80)
    h3 = _layer_call(table3, g3, wd3, ws3, b3a.reshape(1, CH), W3b, bb=b3b.reshape(1, CH), k=3)

    wr_pad = jnp.zeros((CH, 8), jnp.float32).at[:, :6].set(Wr)
    br_pad = jnp.zeros((1, 8), jnp.float32).at[0, :6].set(br)
    out = _head_call(h3, b3pad.reshape(Np3, 1), wr_pad, br_pad)
    return out[:, :6]


# speedup vs baseline: 104.3396x; 1.5007x over previous
import functools

import jax
import jax.numpy as jnp
import numpy as np
from jax import lax
from jax.experimental import pallas as pl
from jax.experimental.pallas import tpu as pltpu
from jax.experimental.pallas import tpu_sc as plsc

CH = 64
NBATCH = 8
PADB = 127
_NW = 32


def _round_up(x, m):
    return (x + m - 1) // m * m


def _sc_gather(table, idx):
    V, D = table.shape
    B = idx.shape[0]
    idx = jnp.clip(idx, 0, V - 1)
    b_per_w = B // _NW
    mesh = plsc.VectorSubcoreMesh(core_axis_name="c", subcore_axis_name="s")

    @functools.partial(
        pl.kernel,
        out_type=jax.ShapeDtypeStruct((B, D), jnp.float32),
        mesh=mesh,
        scratch_types=[
            pltpu.VMEM((b_per_w,), jnp.int32),
            pltpu.VMEM((b_per_w, D), jnp.float32),
            pltpu.SemaphoreType.DMA,
        ],
        compiler_params=pltpu.CompilerParams(use_tc_tiling_on_sc=False),
    )
    def gk(table_hbm, idx_hbm, out_hbm, idx_v, rows_v, sem):
        wid = lax.axis_index("s") * 2 + lax.axis_index("c")
        base = wid * b_per_w
        pltpu.sync_copy(idx_hbm.at[pl.ds(base, b_per_w)], idx_v)
        pltpu.async_copy(table_hbm.at[idx_v], rows_v, sem).wait()
        pltpu.sync_copy(rows_v, out_hbm.at[pl.ds(base, b_per_w)])

    return gk(table, idx)


def _knn_kernel(k, R, Np, pr_ref, br_ref, pT_ref, bc_ref, out_ref):
    colio = lax.broadcasted_iota(jnp.int32, (R, Np), 1)
    d = None
    for c in range(3):
        diff = pr_ref[:, c : c + 1] - pT_ref[c : c + 1, :]
        sq = diff * diff
        d = sq if d is None else d + sq
    inf = jnp.float32(jnp.inf)
    bigf = jnp.float32(3e38)
    same = br_ref[:, 0:1] == bc_ref[...]
    dcur = jnp.where(same, d, bigf)
    big = jnp.int32(2**30)
    for j in range(k):
        mn = jnp.min(dcur, axis=1, keepdims=True)
        amin = jnp.min(
            jnp.where(dcur == mn, colio, big), axis=1, keepdims=True
        )
        out_ref[:, j : j + 1] = amin
        dcur = jnp.where(colio == amin, inf, dcur)


def _knn_call(pos_pad, posT, batch_rows, batch_cols, k, R=256):
    Np = pos_pad.shape[0]
    nt = Np // R
    return pl.pallas_call(
        functools.partial(_knn_kernel, k, R, Np),
        grid=(nt,),
        in_specs=[
            pl.BlockSpec((R, 4), lambda i: (i, 0)),
            pl.BlockSpec((R, 1), lambda i: (i, 0)),
            pl.BlockSpec((4, Np), lambda i: (0, 0)),
            pl.BlockSpec((1, Np), lambda i: (0, 0)),
        ],
        out_specs=pl.BlockSpec((R, 8), lambda i: (i, 0)),
        out_shape=jax.ShapeDtypeStruct((Np, 8), jnp.int32),
    )(pos_pad, batch_rows, posT, batch_cols)


def _fps_kernel(C, Ms, mm_ref, n_ref, px_ref, py_ref, pz_ref, sel_ref):
    px = px_ref[...]
    py = py_ref[...]
    pz = pz_ref[...]
    colio = lax.broadcasted_iota(jnp.int32, (NBATCH, C), 1)
    valid = colio < n_ref[...]
    ninf = jnp.float32(-jnp.inf)
    big = jnp.int32(2**30)
    eye = lax.broadcasted_iota(jnp.int32, (NBATCH, NBATCH), 0) == lax.broadcasted_iota(
        jnp.int32, (NBATCH, NBATCH), 1
    )

    def dist_to(sx, sy, sz):
        dx = px - sx
        dy = py - sy
        dz = pz - sz
        return (dx * dx + dy * dy) + dz * dz

    d0 = dist_to(px[:, 0:1], py[:, 0:1], pz[:, 0:1])
    dd = jnp.where(valid, d0, ninf)
    sel_ref[0:1, :] = jnp.zeros((1, NBATCH), jnp.int32)

    def body(r, dd):
        mx = jnp.max(dd, axis=1, keepdims=True)
        nxt = jnp.min(jnp.where(dd == mx, colio, big), axis=1, keepdims=True)
        row = jnp.sum(
            jnp.where(eye, jnp.broadcast_to(nxt, (NBATCH, NBATCH)), 0),
            axis=0,
            keepdims=True,
        )
        sel_ref[pl.ds(r, 1), :] = row
        hit = colio == nxt
        sx = jnp.max(jnp.where(hit, px, ninf), axis=1, keepdims=True)
        sy = jnp.max(jnp.where(hit, py, ninf), axis=1, keepdims=True)
        sz = jnp.max(jnp.where(hit, pz, ninf), axis=1, keepdims=True)
        return jnp.minimum(dd, dist_to(sx, sy, sz))

    lax.fori_loop(1, mm_ref[0], body, dd)


def _fps_call(px, py, pz, n_seg, max_m, C):
    Ms = C // 2 + 128
    return pl.pallas_call(
        functools.partial(_fps_kernel, C, Ms),
        in_specs=[
            pl.BlockSpec(memory_space=pltpu.SMEM),
            pl.BlockSpec((NBATCH, 1), lambda: (0, 0)),
            pl.BlockSpec((NBATCH, C), lambda: (0, 0)),
            pl.BlockSpec((NBATCH, C), lambda: (0, 0)),
            pl.BlockSpec((NBATCH, C), lambda: (0, 0)),
        ],
        out_specs=pl.BlockSpec((Ms, NBATCH), lambda: (0, 0)),
        out_shape=jax.ShapeDtypeStruct((Ms, NBATCH), jnp.int32),
    )(max_m, n_seg, px, py, pz)


def _layer_kernel(k, tdst_ref, tsrc_ref, wd_ref, ws_ref, ba_ref, wb_ref, bb_ref, out_ref):
    a = (
        jnp.dot(tdst_ref[...], wd_ref[...], preferred_element_type=jnp.float32)
        + ba_ref[...]
    )
    wb = wb_ref[...]
    bb = bb_ref[...]
    acc = None
    for j in range(k):
        bj = jnp.dot(tsrc_ref[j], ws_ref[...], preferred_element_type=jnp.float32)
        m = jnp.dot(jnp.maximum(a + bj, 0.0), wb, preferred_element_type=jnp.float32) + bb
        acc = m if acc is None else jnp.maximum(acc, m)
    out_ref[...] = jnp.maximum(acc, 0.0)


def _layer_call(table, src_gath, wd, ws, ba, wb, bb, k, R=256):
    Np, D = table.shape
    nt = Np // R
    return pl.pallas_call(
        functools.partial(_layer_kernel, k),
        grid=(nt,),
        in_specs=[
            pl.BlockSpec((R, D), lambda i: (i, 0)),
            pl.BlockSpec((k, R, D), lambda i: (0, i, 0)),
            pl.BlockSpec((D, CH), lambda i: (0, 0)),
            pl.BlockSpec((D, CH), lambda i: (0, 0)),
            pl.BlockSpec((1, CH), lambda i: (0, 0)),
            pl.BlockSpec((CH, CH), lambda i: (0, 0)),
            pl.BlockSpec((1, CH), lambda i: (0, 0)),
        ],
        out_specs=pl.BlockSpec((R, CH), lambda i: (i, 0)),
        out_shape=jax.ShapeDtypeStruct((Np, CH), jnp.float32),
    )(table, src_gath, wd, ws, ba, wb, bb)


def _head_kernel(h_ref, b3_ref, wr_ref, br_ref, out_ref):
    ninf = jnp.float32(-jnp.inf)
    h = h_ref[...]
    rows = []
    for b in range(NBATCH):
        mb = b3_ref[...] == b
        rows.append(jnp.max(jnp.where(mb, h, ninf), axis=0, keepdims=True))
    g = jnp.concatenate(rows, axis=0)
    out_ref[...] = jnp.dot(g, wr_ref[...], preferred_element_type=jnp.float32) + br_ref[...]


def _head_call(h3, b3, wr_pad, br_pad):
    Np = h3.shape[0]
    return pl.pallas_call(
        _head_kernel,
        in_specs=[
            pl.BlockSpec((Np, CH), lambda: (0, 0)),
            pl.BlockSpec((Np, 1), lambda: (0, 0)),
            pl.BlockSpec((CH, 8), lambda: (0, 0)),
            pl.BlockSpec((1, 8), lambda: (0, 0)),
        ],
        out_specs=pl.BlockSpec((NBATCH, 8), lambda: (0, 0)),
        out_shape=jax.ShapeDtypeStruct((NBATCH, 8), jnp.float32),
    )(h3, b3, wr_pad, br_pad)


def _pack_weights(wa, hdim, D):
    wd = jnp.zeros((D, CH), jnp.float32)
    ws = jnp.zeros((D, CH), jnp.float32)
    wpos = wa[2 * hdim : 2 * hdim + 3]
    if hdim == 3:
        wd = wd.at[0:3].set(wa[0:3] - wpos)
        ws = ws.at[0:3].set(wa[3:6] + wpos)
    else:
        wd = wd.at[0:CH].set(wa[0:CH]).at[CH : CH + 3].set(-wpos)
        ws = ws.at[0:CH].set(wa[CH : 2 * CH]).at[CH : CH + 3].set(wpos)
    return wd, ws


def _fps_level(posT_seg, seg_start, counts, C, M_pad, C_small):
    m = (counts + 1) // 2
    max_m = jnp.max(m).astype(jnp.int32).reshape((1,))
    Ms_big = C // 2 + 128

    def run(C_):
        wins = jnp.stack(
            [
                lax.dynamic_slice(posT_seg, (0, seg_start[b]), (3, C_))
                for b in range(NBATCH)
            ]
        )
        sel = _fps_call(
            wins[:, 0, :],
            wins[:, 1, :],
            wins[:, 2, :],
            counts.reshape(NBATCH, 1),
            max_m,
            C_,
        )
        if sel.shape[0] < Ms_big:
            sel = jnp.pad(sel, ((0, Ms_big - sel.shape[0]), (0, 0)))
        return sel

    sel = lax.cond(
        jnp.max(counts) <= C_small, lambda: run(C_small), lambda: run(C)
    )
    Ms = sel.shape[0]
    gsel = sel + seg_start[None, :].astype(jnp.int32)
    ar = jnp.arange(Ms, dtype=jnp.int32)
    offs = jnp.cumsum(m) - m
    dest = jnp.where(ar[:, None] < m[None, :], offs[None, :] + ar[:, None], M_pad)
    idx = (
        jnp.zeros((M_pad,), jnp.int32)
        .at[dest.reshape(-1)]
        .set(gsel.reshape(-1), mode="drop")
    )
    return idx, m


def kernel(x, pos, batch, W1a, b1a, W1b, b1b, W2a, b2a, W2b, b2b, W3a, b3a, W3b, b3b, Wr, br):
    N = pos.shape[0]
    Np1 = _round_up(N, 256)
    batch = batch.astype(jnp.int32)

    pos_pad = jnp.zeros((Np1, 4), jnp.float32).at[:N, :3].set(pos)
    posT = pos_pad[:, :3].T
    posT4 = jnp.zeros((4, Np1), jnp.float32).at[:3].set(posT)
    bpad = jnp.full((Np1,), PADB, jnp.int32).at[:N].set(batch)
    counts1 = jnp.sum(
        batch[None, :] == jnp.arange(NBATCH, dtype=jnp.int32)[:, None], axis=1
    ).astype(jnp.int32)
    seg1 = jnp.cumsum(counts1) - counts1

    nbr1 = _knn_call(pos_pad, posT4, bpad.reshape(Np1, 1), bpad.reshape(1, Np1), k=6)
    src1 = nbr1[:, :6].T.reshape(-1)
    table1 = jnp.zeros((Np1, 16), jnp.float32).at[:, :3].set(pos_pad[:, :3])
    g1 = _sc_gather(table1, src1).reshape(6, Np1, 16)
    wd1, ws1 = _pack_weights(W1a, 3, 16)
    h1 = _layer_call(table1, g1, wd1, ws1, b1a.reshape(1, CH), W1b, bb=b1b.reshape(1, CH), k=6)

    posT2x = jnp.zeros((4, 2 * Np1), jnp.float32).at[:, :Np1].set(posT4)
    M_pad1 = (N + 1) // 2 + NBATCH
    i1, m1 = _fps_level(posT2x, seg1, counts1, Np1, M_pad1, C_small=2048)
    Np2 = _round_up(M_pad1, 256)
    i1p = jnp.zeros((Np2,), jnp.int32).at[:M_pad1].set(i1)

    ntab1 = jnp.concatenate([h1, table1], axis=1)
    table2 = _sc_gather(ntab1, i1p)
    pos2 = table2[:, CH : CH + 3]
    cum1 = jnp.cumsum(m1)
    b2 = jnp.searchsorted(cum1, jnp.arange(M_pad1, dtype=jnp.int32), side="right").astype(jnp.int32)
    b2pad = jnp.full((Np2,), PADB, jnp.int32).at[:M_pad1].set(b2)

    pos2_pad = jnp.zeros((Np2, 4), jnp.float32).at[:, :3].set(pos2)
    pos2T4 = jnp.zeros((4, Np2), jnp.float32).at[:3].set(pos2.T)
    nbr2 = _knn_call(pos2_pad, pos2T4, b2pad.reshape(Np2, 1), b2pad.reshape(1, Np2), k=4)
    src2 = nbr2[:, :4].T.reshape(-1)
    g2 = _sc_gather(table2, src2).reshape(4, Np2, 80)
    wd2, ws2 = _pack_weights(W2a, CH, 80)
    h2 = _layer_call(table2, g2, wd2, ws2, b2a.reshape(1, CH), W2b, bb=b2b.reshape(1, CH), k=4)

    pos2T2x = jnp.zeros((4, 2 * Np2), jnp.float32).at[:, :Np2].set(pos2T4)
    M_pad2 = (M_pad1 + 1) // 2 + NBATCH
    i2, m2 = _fps_level(pos2T2x, jnp.cumsum(m1) - m1, m1, Np2, M_pad2, C_small=1024)
    Np3 = _round_up(M_pad2, 256)
    i2p = jnp.zeros((Np3,), jnp.int32).at[:M_pad2].set(i2)

    ntab2 = jnp.concatenate([h2, table2[:, CH : CH + 16]], axis=1)
    table3 = _sc_gather(ntab2, i2p)
    pos3 = table3[:, CH : CH + 3]
    cum2 = jnp.cumsum(m2)
    b3 = jnp.searchsorted(cum2, jnp.arange(M_pad2, dtype=jnp.int32), side="right").astype(jnp.int32)
    b3pad = jnp.full((Np3,), PADB, jnp.int32).at[:M_pad2].set(b3)

    pos3_pad = jnp.zeros((Np3, 4), jnp.float32).at[:, :3].set(pos3)
    pos3T4 = jnp.zeros((4, Np3), jnp.float32).at[:3].set(pos3.T)
    nbr3 = _knn_call(pos3_pad, pos3T4, b3pad.reshape(Np3, 1), b3pad.reshape(1, Np3), k=3)
    src3 = nbr3[:, :3].T.reshape(-1)
    g3 = _sc_gather(table3, src3).reshape(3, Np3, 80)
    wd3, ws3 = _pack_weights(W3a, CH, 80)
    h3 = _layer_call(table3, g3, wd3, ws3, b3a.reshape(1, CH), W3b, bb=b3b.reshape(1, CH), k=3)

    wr_pad = jnp.zeros((CH, 8), jnp.float32).at[:, :6].set(Wr)
    br_pad = jnp.zeros((1, 8), jnp.float32).at[0, :6].set(br)
    out = _head_call(h3, b3pad.reshape(Np3, 1), wr_pad, br_pad)
    return out[:, :6]


# speedup vs baseline: 127.5637x; 1.2226x over previous
import functools

import jax
import jax.numpy as jnp
import numpy as np
from jax import lax
from jax.experimental import pallas as pl
from jax.experimental.pallas import tpu as pltpu
from jax.experimental.pallas import tpu_sc as plsc

CH = 64
NBATCH = 8
PADB = 127
_NW = 32


def _round_up(x, m):
    return (x + m - 1) // m * m


def _sc_gather(table, idx):
    V, D = table.shape
    B = idx.shape[0]
    idx = jnp.clip(idx, 0, V - 1)
    b_per_w = B // _NW
    mesh = plsc.VectorSubcoreMesh(core_axis_name="c", subcore_axis_name="s")

    @functools.partial(
        pl.kernel,
        out_type=jax.ShapeDtypeStruct((B, D), jnp.float32),
        mesh=mesh,
        scratch_types=[
            pltpu.VMEM((b_per_w,), jnp.int32),
            pltpu.VMEM((b_per_w, D), jnp.float32),
            pltpu.SemaphoreType.DMA,
        ],
        compiler_params=pltpu.CompilerParams(use_tc_tiling_on_sc=False),
    )
    def gk(table_hbm, idx_hbm, out_hbm, idx_v, rows_v, sem):
        wid = lax.axis_index("s") * 2 + lax.axis_index("c")
        base = wid * b_per_w
        pltpu.sync_copy(idx_hbm.at[pl.ds(base, b_per_w)], idx_v)
        pltpu.async_copy(table_hbm.at[idx_v], rows_v, sem).wait()
        pltpu.sync_copy(rows_v, out_hbm.at[pl.ds(base, b_per_w)])

    return gk(table, idx)


def _knn_win_kernel(k, R, W, w0_ref, pr_ref, br_ref, pw_ref, bw_ref, out_ref):
    i = pl.program_id(0)
    w0 = w0_ref[i]
    colio = lax.broadcasted_iota(jnp.int32, (R, W), 1)
    pw = pw_ref[0]
    d = None
    for c in range(3):
        diff = pr_ref[:, c : c + 1] - pw[c : c + 1, :]
        sq = diff * diff
        d = sq if d is None else d + sq
    inf = jnp.float32(jnp.inf)
    bigf = jnp.float32(3e38)
    same = br_ref[:, 0:1] == bw_ref[0]
    dcur = jnp.where(same, d, bigf)
    big = jnp.int32(2**30)
    for j in range(k):
        mn = jnp.min(dcur, axis=1, keepdims=True)
        amin = jnp.min(jnp.where(dcur == mn, colio, big), axis=1, keepdims=True)
        out_ref[:, j : j + 1] = amin + w0
        dcur = jnp.where(colio == amin, inf, dcur)


def _knn_kernel(k, R, Np, pr_ref, br_ref, pT_ref, bc_ref, out_ref):
    colio = lax.broadcasted_iota(jnp.int32, (R, Np), 1)
    d = None
    for c in range(3):
        diff = pr_ref[:, c : c + 1] - pT_ref[c : c + 1, :]
        sq = diff * diff
        d = sq if d is None else d + sq
    inf = jnp.float32(jnp.inf)
    bigf = jnp.float32(3e38)
    same = br_ref[:, 0:1] == bc_ref[...]
    dcur = jnp.where(same, d, bigf)
    big = jnp.int32(2**30)
    for j in range(k):
        mn = jnp.min(dcur, axis=1, keepdims=True)
        amin = jnp.min(
            jnp.where(dcur == mn, colio, big), axis=1, keepdims=True
        )
        out_ref[:, j : j + 1] = amin
        dcur = jnp.where(colio == amin, inf, dcur)


def _knn_call(pos_pad, posT, batch_rows, batch_cols, k, R=256):
    Np = pos_pad.shape[0]
    nt = Np // R
    return pl.pallas_call(
        functools.partial(_knn_kernel, k, R, Np),
        grid=(nt,),
        in_specs=[
            pl.BlockSpec((R, 4), lambda i: (i, 0)),
            pl.BlockSpec((R, 1), lambda i: (i, 0)),
            pl.BlockSpec((4, Np), lambda i: (0, 0)),
            pl.BlockSpec((1, Np), lambda i: (0, 0)),
        ],
        out_specs=pl.BlockSpec((R, 8), lambda i: (i, 0)),
        out_shape=jax.ShapeDtypeStruct((Np, 8), jnp.int32),
    )(pos_pad, batch_rows, posT, batch_cols)


def _knn(pos_pad, posT, bpad, k, W, R=256):
    Np = pos_pad.shape[0]
    nt = Np // R
    br = bpad.reshape(Np, 1)
    bc = bpad.reshape(1, Np)
    lo = bpad[::R]
    hi = bpad[R - 1 :: R]
    ws = jnp.searchsorted(bpad, lo, side="left").astype(jnp.int32)
    we = jnp.searchsorted(bpad, hi, side="right").astype(jnp.int32)
    w0 = jnp.minimum(ws // 128 * 128, Np - W).astype(jnp.int32)
    fits = jnp.all(we - w0 <= W)

    def windowed():
        pwins = jnp.stack(
            [lax.dynamic_slice(posT, (0, w0[i]), (4, W)) for i in range(nt)]
        )
        bwins = jnp.stack(
            [lax.dynamic_slice(bc, (0, w0[i]), (1, W)) for i in range(nt)]
        )
        return pl.pallas_call(
            functools.partial(_knn_win_kernel, k, R, W),
            grid=(nt,),
            in_specs=[
                pl.BlockSpec(memory_space=pltpu.SMEM),
                pl.BlockSpec((R, 4), lambda i: (i, 0)),
                pl.BlockSpec((R, 1), lambda i: (i, 0)),
                pl.BlockSpec((1, 4, W), lambda i: (i, 0, 0)),
                pl.BlockSpec((1, 1, W), lambda i: (i, 0, 0)),
            ],
            out_specs=pl.BlockSpec((R, 8), lambda i: (i, 0)),
            out_shape=jax.ShapeDtypeStruct((Np, 8), jnp.int32),
        )(w0, pos_pad, br, pwins, bwins)

    return lax.cond(fits, windowed, lambda: _knn_call(pos_pad, posT, br, bc, k, R))


def _fps_kernel(C, Ms, mm_ref, n_ref, px_ref, py_ref, pz_ref, sel_ref):
    px = px_ref[...]
    py = py_ref[...]
    pz = pz_ref[...]
    colio = lax.broadcasted_iota(jnp.int32, (NBATCH, C), 1)
    valid = colio < n_ref[...]
    ninf = jnp.float32(-jnp.inf)
    big = jnp.int32(2**30)
    eye = lax.broadcasted_iota(jnp.int32, (NBATCH, NBATCH), 0) == lax.broadcasted_iota(
        jnp.int32, (NBATCH, NBATCH), 1
    )

    def dist_to(sx, sy, sz):
        dx = px - sx
        dy = py - sy
        dz = pz - sz
        return (dx * dx + dy * dy) + dz * dz

    d0 = dist_to(px[:, 0:1], py[:, 0:1], pz[:, 0:1])
    dd = jnp.where(valid, d0, ninf)
    sel_ref[0:1, :] = jnp.zeros((1, NBATCH), jnp.int32)

    def body(r, dd):
        mx = jnp.max(dd, axis=1, keepdims=True)
        nxt = jnp.min(jnp.where(dd == mx, colio, big), axis=1, keepdims=True)
        row = jnp.sum(
            jnp.where(eye, jnp.broadcast_to(nxt, (NBATCH, NBATCH)), 0),
            axis=0,
            keepdims=True,
        )
        sel_ref[pl.ds(r, 1), :] = row
        hit = colio == nxt
        sx = jnp.max(jnp.where(hit, px, ninf), axis=1, keepdims=True)
        sy = jnp.max(jnp.where(hit, py, ninf), axis=1, keepdims=True)
        sz = jnp.max(jnp.where(hit, pz, ninf), axis=1, keepdims=True)
        return jnp.minimum(dd, dist_to(sx, sy, sz))

    lax.fori_loop(1, mm_ref[0], body, dd)


def _fps_call(px, py, pz, n_seg, max_m, C):
    Ms = C // 2 + 128
    return pl.pallas_call(
        functools.partial(_fps_kernel, C, Ms),
        in_specs=[
            pl.BlockSpec(memory_space=pltpu.SMEM),
            pl.BlockSpec((NBATCH, 1), lambda: (0, 0)),
            pl.BlockSpec((NBATCH, C), lambda: (0, 0)),
            pl.BlockSpec((NBATCH, C), lambda: (0, 0)),
            pl.BlockSpec((NBATCH, C), lambda: (0, 0)),
        ],
        out_specs=pl.BlockSpec((Ms, NBATCH), lambda: (0, 0)),
        out_shape=jax.ShapeDtypeStruct((Ms, NBATCH), jnp.int32),
    )(max_m, n_seg, px, py, pz)


def _layer_kernel(k, tdst_ref, tsrc_ref, wd_ref, ws_ref, ba_ref, wb_ref, bb_ref, out_ref):
    a = (
        jnp.dot(tdst_ref[...], wd_ref[...], preferred_element_type=jnp.float32)
        + ba_ref[...]
    )
    wb = wb_ref[...]
    bb = bb_ref[...]
    acc = None
    for j in range(k):
        bj = jnp.dot(tsrc_ref[j], ws_ref[...], preferred_element_type=jnp.float32)
        m = jnp.dot(jnp.maximum(a + bj, 0.0), wb, preferred_element_type=jnp.float32) + bb
        acc = m if acc is None else jnp.maximum(acc, m)
    out_ref[...] = jnp.maximum(acc, 0.0)


def _layer_call(table, src_gath, wd, ws, ba, wb, bb, k, R=256):
    Np, D = table.shape
    nt = Np // R
    return pl.pallas_call(
        functools.partial(_layer_kernel, k),
        grid=(nt,),
        in_specs=[
            pl.BlockSpec((R, D), lambda i: (i, 0)),
            pl.BlockSpec((k, R, D), lambda i: (0, i, 0)),
            pl.BlockSpec((D, CH), lambda i: (0, 0)),
            pl.BlockSpec((D, CH), lambda i: (0, 0)),
            pl.BlockSpec((1, CH), lambda i: (0, 0)),
            pl.BlockSpec((CH, CH), lambda i: (0, 0)),
            pl.BlockSpec((1, CH), lambda i: (0, 0)),
        ],
        out_specs=pl.BlockSpec((R, CH), lambda i: (i, 0)),
        out_shape=jax.ShapeDtypeStruct((Np, CH), jnp.float32),
    )(table, src_gath, wd, ws, ba, wb, bb)


def _head_kernel(h_ref, b3_ref, wr_ref, br_ref, out_ref):
    ninf = jnp.float32(-jnp.inf)
    h = h_ref[...]
    rows = []
    for b in range(NBATCH):
        mb = b3_ref[...] == b
        rows.append(jnp.max(jnp.where(mb, h, ninf), axis=0, keepdims=True))
    g = jnp.concatenate(rows, axis=0)
    out_ref[...] = jnp.dot(g, wr_ref[...], preferred_element_type=jnp.float32) + br_ref[...]


def _head_call(h3, b3, wr_pad, br_pad):
    Np = h3.shape[0]
    return pl.pallas_call(
        _head_kernel,
        in_specs=[
            pl.BlockSpec((Np, CH), lambda: (0, 0)),
            pl.BlockSpec((Np, 1), lambda: (0, 0)),
            pl.BlockSpec((CH, 8), lambda: (0, 0)),
            pl.BlockSpec((1, 8), lambda: (0, 0)),
        ],
        out_specs=pl.BlockSpec((NBATCH, 8), lambda: (0, 0)),
        out_shape=jax.ShapeDtypeStruct((NBATCH, 8), jnp.float32),
    )(h3, b3, wr_pad, br_pad)


def _pack_weights(wa, hdim, D):
    wd = jnp.zeros((D, CH), jnp.float32)
    ws = jnp.zeros((D, CH), jnp.float32)
    wpos = wa[2 * hdim : 2 * hdim + 3]
    if hdim == 3:
        wd = wd.at[0:3].set(wa[0:3] - wpos)
        ws = ws.at[0:3].set(wa[3:6] + wpos)
    else:
        wd = wd.at[0:CH].set(wa[0:CH]).at[CH : CH + 3].set(-wpos)
        ws = ws.at[0:CH].set(wa[CH : 2 * CH]).at[CH : CH + 3].set(wpos)
    return wd, ws


def _fps_level(posT_seg, seg_start, counts, C, M_pad, C_small):
    m = (counts + 1) // 2
    max_m = jnp.max(m).astype(jnp.int32).reshape((1,))
    Ms_big = C // 2 + 128

    def run(C_):
        wins = jnp.stack(
            [
                lax.dynamic_slice(posT_seg, (0, seg_start[b]), (3, C_))
                for b in range(NBATCH)
            ]
        )
        sel = _fps_call(
            wins[:, 0, :],
            wins[:, 1, :],
            wins[:, 2, :],
            counts.reshape(NBATCH, 1),
            max_m,
            C_,
        )
        if sel.shape[0] < Ms_big:
            sel = jnp.pad(sel, ((0, Ms_big - sel.shape[0]), (0, 0)))
        return sel

    sel = lax.cond(
        jnp.max(counts) <= C_small, lambda: run(C_small), lambda: run(C)
    )
    Ms = sel.shape[0]
    gsel = sel + seg_start[None, :].astype(jnp.int32)
    ar = jnp.arange(Ms, dtype=jnp.int32)
    offs = jnp.cumsum(m) - m
    dest = jnp.where(ar[:, None] < m[None, :], offs[None, :] + ar[:, None], M_pad)
    idx = (
        jnp.zeros((M_pad,), jnp.int32)
        .at[dest.reshape(-1)]
        .set(gsel.reshape(-1), mode="drop")
    )
    return idx, m


def kernel(x, pos, batch, W1a, b1a, W1b, b1b, W2a, b2a, W2b, b2b, W3a, b3a, W3b, b3b, Wr, br):
    N = pos.shape[0]
    Np1 = _round_up(N, 256)
    batch = batch.astype(jnp.int32)

    pos_pad = jnp.zeros((Np1, 4), jnp.float32).at[:N, :3].set(pos)
    posT = pos_pad[:, :3].T
    posT4 = jnp.zeros((4, Np1), jnp.float32).at[:3].set(posT)
    bpad = jnp.full((Np1,), PADB, jnp.int32).at[:N].set(batch)
    counts1 = jnp.sum(
        batch[None, :] == jnp.arange(NBATCH, dtype=jnp.int32)[:, None], axis=1
    ).astype(jnp.int32)
    seg1 = jnp.cumsum(counts1) - counts1

    nbr1 = _knn(pos_pad, posT4, bpad, k=6, W=3072)
    src1 = nbr1[:, :6].T.reshape(-1)
    table1 = jnp.zeros((Np1, 16), jnp.float32).at[:, :3].set(pos_pad[:, :3])
    g1 = _sc_gather(table1, src1).reshape(6, Np1, 16)
    wd1, ws1 = _pack_weights(W1a, 3, 16)
    h1 = _layer_call(table1, g1, wd1, ws1, b1a.reshape(1, CH), W1b, bb=b1b.reshape(1, CH), k=6)

    posT2x = jnp.zeros((4, 2 * Np1), jnp.float32).at[:, :Np1].set(posT4)
    M_pad1 = (N + 1) // 2 + NBATCH
    i1, m1 = _fps_level(posT2x, seg1, counts1, Np1, M_pad1, C_small=2048)
    Np2 = _round_up(M_pad1, 256)
    i1p = jnp.zeros((Np2,), jnp.int32).at[:M_pad1].set(i1)

    ntab1 = jnp.concatenate([h1, table1], axis=1)
    table2 = _sc_gather(ntab1, i1p)
    pos2 = table2[:, CH : CH + 3]
    cum1 = jnp.cumsum(m1)
    b2 = jnp.searchsorted(cum1, jnp.arange(M_pad1, dtype=jnp.int32), side="right").astype(jnp.int32)
    b2pad = jnp.full((Np2,), PADB, jnp.int32).at[:M_pad1].set(b2)

    pos2_pad = jnp.zeros((Np2, 4), jnp.float32).at[:, :3].set(pos2)
    pos2T4 = jnp.zeros((4, Np2), jnp.float32).at[:3].set(pos2.T)
    nbr2 = _knn(pos2_pad, pos2T4, b2pad, k=4, W=2048)
    src2 = nbr2[:, :4].T.reshape(-1)
    g2 = _sc_gather(table2, src2).reshape(4, Np2, 80)
    wd2, ws2 = _pack_weights(W2a, CH, 80)
    h2 = _layer_call(table2, g2, wd2, ws2, b2a.reshape(1, CH), W2b, bb=b2b.reshape(1, CH), k=4)

    pos2T2x = jnp.zeros((4, 2 * Np2), jnp.float32).at[:, :Np2].set(pos2T4)
    M_pad2 = (M_pad1 + 1) // 2 + NBATCH
    i2, m2 = _fps_level(pos2T2x, jnp.cumsum(m1) - m1, m1, Np2, M_pad2, C_small=1024)
    Np3 = _round_up(M_pad2, 256)
    i2p = jnp.zeros((Np3,), jnp.int32).at[:M_pad2].set(i2)

    ntab2 = jnp.concatenate([h2, table2[:, CH : CH + 16]], axis=1)
    table3 = _sc_gather(ntab2, i2p)
    pos3 = table3[:, CH : CH + 3]
    cum2 = jnp.cumsum(m2)
    b3 = jnp.searchsorted(cum2, jnp.arange(M_pad2, dtype=jnp.int32), side="right").astype(jnp.int32)
    b3pad = jnp.full((Np3,), PADB, jnp.int32).at[:M_pad2].set(b3)

    pos3_pad = jnp.zeros((Np3, 4), jnp.float32).at[:, :3].set(pos3)
    pos3T4 = jnp.zeros((4, Np3), jnp.float32).at[:3].set(pos3.T)
    nbr3 = _knn(pos3_pad, pos3T4, b3pad, k=3, W=1024)
    src3 = nbr3[:, :3].T.reshape(-1)
    g3 = _sc_gather(table3, src3).reshape(3, Np3, 80)
    wd3, ws3 = _pack_weights(W3a, CH, 80)
    h3 = _layer_call(table3, g3, wd3, ws3, b3a.reshape(1, CH), W3b, bb=b3b.reshape(1, CH), k=3)

    wr_pad = jnp.zeros((CH, 8), jnp.float32).at[:, :6].set(Wr)
    br_pad = jnp.zeros((1, 8), jnp.float32).at[0, :6].set(br)
    out = _head_call(h3, b3pad.reshape(Np3, 1), wr_pad, br_pad)
    return out[:, :6]
